# baseline (speedup 1.0000x reference)
import functools

import jax
import jax.numpy as jnp
from jax import lax
from jax.experimental import pallas as pl
from jax.experimental.pallas import tpu as pltpu
from jax.experimental.pallas import tpu_sc as plsc

C, H, W = 96, 512, 512
HW = H * W
NBINS = 256

NC = 2
NS = 16
NW = NC * NS
CPW = C // NW
CH = 16384
NCHUNK = HW // CH
HSTRIDE = NBINS + 1
PROBE = "off"


def _stats_body(t_ref, mj_ref, mi_ref, out_ref):
    t = t_ref[0]
    m = mj_ref[...]
    big = jnp.float32(1e30)
    minJ = jnp.min(jnp.where(m > 0, t, big))
    maxJ = jnp.max(jnp.where(m > 0, t, -big))
    nI = jnp.sum(mi_ref[...])
    nJ = jnp.sum(m)
    rng = jnp.maximum(maxJ - minJ, jnp.float32(1e-8))
    inv = jnp.float32(NBINS) / rng
    scale = nI / nJ
    lanes = lax.broadcasted_iota(jnp.int32, (1, 1, 128), 2)
    row = jnp.where(lanes == 0, minJ, jnp.where(lanes == 1, inv, scale))
    out_ref[...] = row


def _stats(t3, mJ, mI):
    return pl.pallas_call(
        _stats_body,
        grid=(C,),
        in_specs=[
            pl.BlockSpec((1, H, W), lambda c: (c, 0, 0)),
            pl.BlockSpec((H, W), lambda c: (0, 0)),
            pl.BlockSpec((H, W), lambda c: (0, 0)),
        ],
        out_specs=pl.BlockSpec((1, 1, 128), lambda c: (c, 0, 0)),
        out_shape=jax.ShapeDtypeStruct((C, 1, 128), jnp.float32),
    )(t3, mJ, mI)


def _sc_body(stats_hbm, tgt_hbm, mj_hbm, out_hbm,
             tbuf, mbuf, hist0, hist1, cumbuf, statbuf,
             sem_t0, sem_t1, sem_m0, sem_m1):
    hists = (hist0, hist1)
    wid = lax.axis_index("s") * NC + lax.axis_index("c")
    lane = lax.iota(jnp.int32, 16)
    lane_base = lane * HSTRIDE
    ones16 = jnp.ones((16,), jnp.float32)
    zeros16 = jnp.zeros((16,), jnp.float32)
    sem_t = (sem_t0, sem_t1)
    sem_m = (sem_m0, sem_m1)

    for k in range(CPW):
        ch = wid * CPW + k
        pltpu.sync_copy(stats_hbm.at[ch], statbuf)
        statv = statbuf[pl.ds(0, 16)]
        mn = statv[0]
        inv = statv[1]
        scale = statv[2]

        for h in hists:
            def zero_body(i, _, h=h):
                h[pl.ds(i * 16, 16)] = zeros16
                return 0
            lax.fori_loop(0, (16 * HSTRIDE + 16) // 16, zero_body, 0)

        cp_t = [None, None]
        cp_m = [None, None]
        cp_t[0] = pltpu.async_copy(tgt_hbm.at[ch, pl.ds(0, CH)],
                                   tbuf.at[0], sem_t[0])
        cp_m[0] = pltpu.async_copy(mj_hbm.at[pl.ds(0, CH)],
                                   mbuf.at[0], sem_m[0])
        for j in range(NCHUNK):
            p = j % 2
            if j + 1 < NCHUNK:
                q = (j + 1) % 2
                cp_t[q] = pltpu.async_copy(
                    tgt_hbm.at[ch, pl.ds((j + 1) * CH, CH)], tbuf.at[q],
                    sem_t[q])
                cp_m[q] = pltpu.async_copy(
                    mj_hbm.at[pl.ds((j + 1) * CH, CH)], mbuf.at[q],
                    sem_m[q])
            cp_t[p].wait()
            cp_m[p].wait()

            def inner(base):
                for u in range(2):
                    off = base + u * 16
                    t = tbuf[p, pl.ds(off, 16)]
                    m = mbuf[p, pl.ds(off, 16)]
                    s = (t - mn) * inv
                    s = jnp.minimum(jnp.maximum(s, jnp.float32(0.0)),
                                    jnp.float32(NBINS - 1))
                    idx = s.astype(jnp.int32) + lane_base
                    plsc.addupdate_scatter(hists[u], [idx], ones16,
                                           mask=m > jnp.float32(0.0))
            if PROBE != "dma":
                plsc.parallel_loop(0, CH, step=32, unroll=2)(inner)

        def red_body(g, carry):
            acc = zeros16
            for h in hists:
                for l in range(16):
                    acc = acc + h[pl.ds(l * HSTRIDE + g * 16, 16)]
            cs = plsc.cumsum(acc) + carry
            cumbuf[pl.ds(g * 16, 16)] = cs
            return carry + jnp.sum(acc)
        lax.fori_loop(0, NBINS // 16, red_body, jnp.float32(0.0))

        def scale_body(g, _):
            cumbuf[pl.ds(g * 16, 16)] = cumbuf[pl.ds(g * 16, 16)] * scale
            return 0
        lax.fori_loop(0, NBINS // 16, scale_body, 0)

        pltpu.sync_copy(cumbuf, out_hbm.at[ch])


def _sc_hist(stats, t2, mJ1):
    mesh = plsc.VectorSubcoreMesh(core_axis_name="c", subcore_axis_name="s")
    f = functools.partial(
        pl.kernel,
        mesh=mesh,
        compiler_params=pltpu.CompilerParams(needs_layout_passes=False,
                                             use_tc_tiling_on_sc=True),
        out_type=jax.ShapeDtypeStruct((C, NBINS), jnp.float32),
        scratch_types=[
            pltpu.VMEM((2, CH), jnp.float32),
            pltpu.VMEM((2, CH), jnp.float32),
            pltpu.VMEM((16 * HSTRIDE + 16,), jnp.float32),
            pltpu.VMEM((16 * HSTRIDE + 16,), jnp.float32),
            pltpu.VMEM((NBINS,), jnp.float32),
            pltpu.VMEM((128,), jnp.float32),
            pltpu.SemaphoreType.DMA,
            pltpu.SemaphoreType.DMA,
            pltpu.SemaphoreType.DMA,
            pltpu.SemaphoreType.DMA,
        ],
    )(_sc_body)
    return f(stats, t2, mJ1)


def kernel(input, target, maskI, maskJ, mask):
    t3 = target.reshape(C, H, W)
    mJ = maskJ.reshape(H, W).astype(jnp.float32)
    mI = maskI.reshape(H, W).astype(jnp.float32)
    stats = _stats(t3, mJ, mI).reshape(C, 128)
    cumJ = _sc_hist(stats, target.reshape(C, HW), mJ.reshape(HW))
    return (input, cumJ)


# speedup vs baseline: 1.1794x; 1.1794x over previous
import functools

import jax
import jax.numpy as jnp
from jax import lax
from jax.experimental import pallas as pl
from jax.experimental.pallas import tpu as pltpu
from jax.experimental.pallas import tpu_sc as plsc

C, H, W = 96, 512, 512
HW = H * W
NBINS = 256

NC = 2
NS = 16
NW = NC * NS
CPW = C // NW
CH = 16384
NCHUNK = HW // CH
HSTRIDE = NBINS + 1
PROBE = "off"


CGRP = 8


def _stats_body(t_ref, mj_ref, mi_ref, out_ref):
    m = mj_ref[...]
    big = jnp.float32(1e30)
    nI = jnp.sum(mi_ref[...])
    nJ = jnp.sum(m)
    scale = nI / nJ
    lanes = lax.broadcasted_iota(jnp.int32, (1, 1, 128), 2)
    rows = []
    for c in range(CGRP):
        t = t_ref[c]
        minJ = jnp.min(jnp.where(m > 0, t, big))
        maxJ = jnp.max(jnp.where(m > 0, t, -big))
        rng = jnp.maximum(maxJ - minJ, jnp.float32(1e-8))
        inv = jnp.float32(NBINS) / rng
        offs = -minJ * inv
        rows.append(jnp.where(lanes == 0, offs,
                              jnp.where(lanes == 1, inv, scale)))
    out_ref[...] = jnp.concatenate(rows, axis=0)


def _stats(t3, mJ, mI):
    return pl.pallas_call(
        _stats_body,
        grid=(C // CGRP,),
        in_specs=[
            pl.BlockSpec((CGRP, H, W), lambda c: (c, 0, 0)),
            pl.BlockSpec((H, W), lambda c: (0, 0)),
            pl.BlockSpec((H, W), lambda c: (0, 0)),
        ],
        out_specs=pl.BlockSpec((CGRP, 1, 128), lambda c: (c, 0, 0)),
        out_shape=jax.ShapeDtypeStruct((C, 1, 128), jnp.float32),
    )(t3, mJ, mI)


def _copy_body(in_ref, out_ref):
    out_ref[...] = in_ref[...]


def _passthrough(x):
    return pl.pallas_call(
        _copy_body,
        grid=(C // CGRP,),
        in_specs=[pl.BlockSpec((1, CGRP, H, W), lambda c: (0, c, 0, 0))],
        out_specs=pl.BlockSpec((1, CGRP, H, W), lambda c: (0, c, 0, 0)),
        out_shape=jax.ShapeDtypeStruct(x.shape, x.dtype),
    )(x)


def _sc_body(stats_hbm, tgt_hbm, mj_hbm, out_hbm,
             tbuf, mbuf, hist0, hist1, cumbuf, statbuf,
             sem_t0, sem_t1, sem_m0, sem_m1):
    hists = (hist0, hist1)
    wid = lax.axis_index("s") * NC + lax.axis_index("c")
    lane = lax.iota(jnp.int32, 16)
    lane_base = lane * HSTRIDE
    ones16 = jnp.ones((16,), jnp.float32)
    zeros16 = jnp.zeros((16,), jnp.float32)
    sem_t = (sem_t0, sem_t1)
    sem_m = (sem_m0, sem_m1)

    for k in range(CPW):
        ch = wid * CPW + k
        pltpu.sync_copy(stats_hbm.at[ch], statbuf)
        statv = statbuf[pl.ds(0, 16)]
        offs = statv[0]
        inv = statv[1]
        scale = statv[2]

        for h in hists:
            def zero_body(i, _, h=h):
                h[pl.ds(i * 16, 16)] = zeros16
                return 0
            lax.fori_loop(0, (16 * HSTRIDE + 16) // 16, zero_body, 0)

        cp_t = [None, None]
        cp_m = [None, None]
        cp_t[0] = pltpu.async_copy(tgt_hbm.at[ch, pl.ds(0, CH)],
                                   tbuf.at[0], sem_t[0])
        cp_m[0] = pltpu.async_copy(mj_hbm.at[pl.ds(0, CH)],
                                   mbuf.at[0], sem_m[0])
        for j in range(NCHUNK):
            p = j % 2
            if j + 1 < NCHUNK:
                q = (j + 1) % 2
                cp_t[q] = pltpu.async_copy(
                    tgt_hbm.at[ch, pl.ds((j + 1) * CH, CH)], tbuf.at[q],
                    sem_t[q])
                cp_m[q] = pltpu.async_copy(
                    mj_hbm.at[pl.ds((j + 1) * CH, CH)], mbuf.at[q],
                    sem_m[q])
            cp_t[p].wait()
            cp_m[p].wait()

            def inner(base):
                for u in range(2):
                    off = base + u * 16
                    t = tbuf[p, pl.ds(off, 16)]
                    m = mbuf[p, pl.ds(off, 16)]
                    s = t * inv + offs
                    s = jnp.minimum(jnp.maximum(s, jnp.float32(0.0)),
                                    jnp.float32(NBINS - 1))
                    idx = s.astype(jnp.int32) + lane_base
                    plsc.addupdate_scatter(hists[u], [idx], ones16,
                                           mask=m > jnp.float32(0.0))
            if PROBE != "dma":
                plsc.parallel_loop(0, CH, step=32, unroll=2)(inner)

        def red_body(g, carry):
            acc = zeros16
            for h in hists:
                for l in range(16):
                    acc = acc + h[pl.ds(l * HSTRIDE + g * 16, 16)]
            cs = plsc.cumsum(acc) + carry
            cumbuf[pl.ds(g * 16, 16)] = cs
            return carry + jnp.sum(acc)
        lax.fori_loop(0, NBINS // 16, red_body, jnp.float32(0.0))

        def scale_body(g, _):
            cumbuf[pl.ds(g * 16, 16)] = cumbuf[pl.ds(g * 16, 16)] * scale
            return 0
        lax.fori_loop(0, NBINS // 16, scale_body, 0)

        pltpu.sync_copy(cumbuf, out_hbm.at[ch])


def _sc_hist(stats, t2, mJ1):
    mesh = plsc.VectorSubcoreMesh(core_axis_name="c", subcore_axis_name="s")
    f = functools.partial(
        pl.kernel,
        mesh=mesh,
        compiler_params=pltpu.CompilerParams(needs_layout_passes=False,
                                             use_tc_tiling_on_sc=True),
        out_type=jax.ShapeDtypeStruct((C, NBINS), jnp.float32),
        scratch_types=[
            pltpu.VMEM((2, CH), jnp.float32),
            pltpu.VMEM((2, CH), jnp.float32),
            pltpu.VMEM((16 * HSTRIDE + 16,), jnp.float32),
            pltpu.VMEM((16 * HSTRIDE + 16,), jnp.float32),
            pltpu.VMEM((NBINS,), jnp.float32),
            pltpu.VMEM((128,), jnp.float32),
            pltpu.SemaphoreType.DMA,
            pltpu.SemaphoreType.DMA,
            pltpu.SemaphoreType.DMA,
            pltpu.SemaphoreType.DMA,
        ],
    )(_sc_body)
    return f(stats, t2, mJ1)


def kernel(input, target, maskI, maskJ, mask):
    t3 = target.reshape(C, H, W)
    mJ = maskJ.reshape(H, W).astype(jnp.float32)
    mI = maskI.reshape(H, W).astype(jnp.float32)
    out = _passthrough(input)
    stats = _stats(t3, mJ, mI).reshape(C, 128)
    cumJ = _sc_hist(stats, target.reshape(C, HW), mJ.reshape(HW))
    return (out, cumJ)


# speedup vs baseline: 1.2751x; 1.0811x over previous
import functools

import jax
import jax.numpy as jnp
from jax import lax
from jax.experimental import pallas as pl
from jax.experimental.pallas import tpu as pltpu
from jax.experimental.pallas import tpu_sc as plsc

C, H, W = 96, 512, 512
HW = H * W
NBINS = 256

NC = 2
NS = 16
NW = NC * NS
CPW = C // NW
CH = 32768
NCHUNK = HW // CH
HSTRIDE = NBINS + 1
PROBE = "off"


CGRP = 8


def _stats_body(t_ref, mj_ref, mi_ref, out_ref):
    m = mj_ref[...]
    big = jnp.float32(1e30)
    nI = jnp.sum(mi_ref[...])
    nJ = jnp.sum(m)
    scale = nI / nJ
    lanes = lax.broadcasted_iota(jnp.int32, (1, 1, 128), 2)
    rows = []
    for c in range(CGRP):
        t = t_ref[c]
        minJ = jnp.min(jnp.where(m > 0, t, big))
        maxJ = jnp.max(jnp.where(m > 0, t, -big))
        rng = jnp.maximum(maxJ - minJ, jnp.float32(1e-8))
        inv = jnp.float32(NBINS) / rng
        offs = -minJ * inv
        rows.append(jnp.where(lanes == 0, offs,
                              jnp.where(lanes == 1, inv, scale)))
    out_ref[...] = jnp.concatenate(rows, axis=0)


def _stats(t3, mJ, mI):
    return pl.pallas_call(
        _stats_body,
        grid=(C // CGRP,),
        in_specs=[
            pl.BlockSpec((CGRP, H, W), lambda c: (c, 0, 0)),
            pl.BlockSpec((H, W), lambda c: (0, 0)),
            pl.BlockSpec((H, W), lambda c: (0, 0)),
        ],
        out_specs=pl.BlockSpec((CGRP, 1, 128), lambda c: (c, 0, 0)),
        out_shape=jax.ShapeDtypeStruct((C, 1, 128), jnp.float32),
    )(t3, mJ, mI)


def _copy_body(in_ref, out_ref):
    out_ref[...] = in_ref[...]


def _passthrough(x):
    return pl.pallas_call(
        _copy_body,
        grid=(C // CGRP,),
        in_specs=[pl.BlockSpec((1, CGRP, H, W), lambda c: (0, c, 0, 0))],
        out_specs=pl.BlockSpec((1, CGRP, H, W), lambda c: (0, c, 0, 0)),
        out_shape=jax.ShapeDtypeStruct(x.shape, x.dtype),
    )(x)


def _sc_body(stats_hbm, tgt_hbm, mj_hbm, out_hbm,
             tbuf, mbuf, hist0, hist1, cumbuf, statbuf,
             sem_t0, sem_t1, sem_m0, sem_m1):
    hists = (hist0, hist1)
    wid = lax.axis_index("s") * NC + lax.axis_index("c")
    lane = lax.iota(jnp.int32, 16)
    lane_base = lane * HSTRIDE
    ones16 = jnp.ones((16,), jnp.float32)
    zeros16 = jnp.zeros((16,), jnp.float32)
    sem_t = (sem_t0, sem_t1)
    sem_m = (sem_m0, sem_m1)

    for k in range(CPW):
        ch = wid * CPW + k
        pltpu.sync_copy(stats_hbm.at[ch], statbuf)
        statv = statbuf[pl.ds(0, 16)]
        offs = statv[0]
        inv = statv[1]
        scale = statv[2]

        for h in hists:
            def zero_body(i, _, h=h):
                h[pl.ds(i * 16, 16)] = zeros16
                return 0
            lax.fori_loop(0, (16 * HSTRIDE + 16) // 16, zero_body, 0)

        cp_t = [None, None]
        cp_m = [None, None]
        cp_t[0] = pltpu.async_copy(tgt_hbm.at[ch, pl.ds(0, CH)],
                                   tbuf.at[0], sem_t[0])
        cp_m[0] = pltpu.async_copy(mj_hbm.at[pl.ds(0, CH // 4)],
                                   mbuf.at[0], sem_m[0])
        for j in range(NCHUNK):
            p = j % 2
            if j + 1 < NCHUNK:
                q = (j + 1) % 2
                cp_t[q] = pltpu.async_copy(
                    tgt_hbm.at[ch, pl.ds((j + 1) * CH, CH)], tbuf.at[q],
                    sem_t[q])
                cp_m[q] = pltpu.async_copy(
                    mj_hbm.at[pl.ds((j + 1) * (CH // 4), CH // 4)],
                    mbuf.at[q], sem_m[q])
            cp_t[p].wait()
            cp_m[p].wait()

            def inner(base):
                m32 = mbuf[p, pl.ds(base // 4, 16)]
                for u in range(4):
                    off = base + u * 16
                    t = tbuf[p, pl.ds(off, 16)]
                    mu = (m32 >> (8 * u)) & jnp.int32(255)
                    s = t * inv + offs
                    s = jnp.minimum(jnp.maximum(s, jnp.float32(0.0)),
                                    jnp.float32(NBINS - 1))
                    idx = s.astype(jnp.int32) + lane_base
                    plsc.addupdate_scatter(hists[u % 2], [idx], ones16,
                                           mask=mu > 0)
            if PROBE != "dma":
                plsc.parallel_loop(0, CH, step=64, unroll=1)(inner)

        def red_body(g, carry):
            acc = zeros16
            for h in hists:
                for l in range(16):
                    acc = acc + h[pl.ds(l * HSTRIDE + g * 16, 16)]
            cs = plsc.cumsum(acc) + carry
            cumbuf[pl.ds(g * 16, 16)] = cs
            return carry + jnp.sum(acc)
        lax.fori_loop(0, NBINS // 16, red_body, jnp.float32(0.0))

        def scale_body(g, _):
            cumbuf[pl.ds(g * 16, 16)] = cumbuf[pl.ds(g * 16, 16)] * scale
            return 0
        lax.fori_loop(0, NBINS // 16, scale_body, 0)

        pltpu.sync_copy(cumbuf, out_hbm.at[ch])


def _sc_hist(stats, t2, mJ1):
    mesh = plsc.VectorSubcoreMesh(core_axis_name="c", subcore_axis_name="s")
    f = functools.partial(
        pl.kernel,
        mesh=mesh,
        compiler_params=pltpu.CompilerParams(needs_layout_passes=False,
                                             use_tc_tiling_on_sc=True),
        out_type=jax.ShapeDtypeStruct((C, NBINS), jnp.float32),
        scratch_types=[
            pltpu.VMEM((2, CH), jnp.float32),
            pltpu.VMEM((2, CH // 4), jnp.int32),
            pltpu.VMEM((16 * HSTRIDE + 16,), jnp.float32),
            pltpu.VMEM((16 * HSTRIDE + 16,), jnp.float32),
            pltpu.VMEM((NBINS,), jnp.float32),
            pltpu.VMEM((128,), jnp.float32),
            pltpu.SemaphoreType.DMA,
            pltpu.SemaphoreType.DMA,
            pltpu.SemaphoreType.DMA,
            pltpu.SemaphoreType.DMA,
        ],
    )(_sc_body)
    return f(stats, t2, mJ1)


def kernel(input, target, maskI, maskJ, mask):
    t3 = target.reshape(C, H, W)
    mJ = maskJ.reshape(H, W).astype(jnp.float32)
    mI = maskI.reshape(H, W).astype(jnp.float32)
    out = _passthrough(input)
    stats = _stats(t3, mJ, mI).reshape(C, 128)
    mj8 = maskJ.reshape(HW // 64, 4, 16).swapaxes(1, 2).astype(jnp.int8)
    mj_packed = lax.bitcast_convert_type(mj8.reshape(HW // 4, 4), jnp.int32)
    cumJ = _sc_hist(stats, target.reshape(C, HW), mj_packed)
    return (out, cumJ)
